# Initial kernel scaffold; baseline (speedup 1.0000x reference)
import jax
import jax.numpy as jnp
from jax.experimental import pallas as pl


def kernel(x, w1, b1, w2, b2, w3, b3, w4, b4, wl, bl):
    raise NotImplementedError("write your pallas kernel here")


# speedup vs baseline: 1.1489x; 1.1489x over previous
import jax
import jax.numpy as jnp
from jax.experimental import pallas as pl
from jax.experimental.pallas import tpu as pltpu

H, W = 80, 64
NI = 8

S1 = 1280
S2, R2 = 20 * 18, 22 * 18 + 2
S3, R3 = 10 * 10, 12 * 10 + 2
S4, R4 = 5 * 6, 7 * 6 + 2


def _pool4(y, cout):
    return jnp.maximum(
        jnp.maximum(y[:, 0 * cout:1 * cout], y[:, 1 * cout:2 * cout]),
        jnp.maximum(y[:, 2 * cout:3 * cout], y[:, 3 * cout:4 * cout]))


def _repack(p_ref, a_ref, hn, wn, wp_src, c):
    a_ref[...] = jnp.zeros_like(a_ref)
    wp = wn + 2
    for sy in range(2):
        for sx in range(2):
            lo = (2 * sy + sx) * c
            for h in range(hn):
                src = p_ref[:, pl.ds((2 * h + sy) * wp_src + sx, wn, 2), :]
                a_ref[:, pl.ds((h + 1) * wp + 1, wn), lo:lo + c] = (
                    src.astype(a_ref.dtype))


def _cnn_kernel(x_ref, w1_ref, b1_ref, w2_ref, b2_ref, w3_ref, b3_ref,
                w4_ref, b4_ref, wl_ref, bl_ref, o_ref,
                p1_ref, a2_ref, p2_ref, a3_ref, p3_ref, a4_ref):
    def conv_img(a_ref, i, s, wp, w_ref, b_ref, cout):
        acc = None
        for dy in range(3):
            xc = jnp.concatenate(
                [a_ref[i, pl.ds(dy * wp + dx, s), :] for dx in range(3)],
                axis=1)
            d = jnp.dot(xc, w_ref[dy], preferred_element_type=jnp.float32)
            acc = d if acc is None else acc + d
        y = jnp.maximum(acc + b_ref[...], 0.0)
        return _pool4(y, cout)

    for i in range(NI):
        for m0 in range(0, S1, 640):
            acc = jnp.dot(x_ref[i, pl.ds(m0, 640), :], w1_ref[...],
                          preferred_element_type=jnp.float32)
            y = jnp.maximum(acc + b1_ref[...], 0.0)
            p1_ref[i, pl.ds(m0, 640), :] = _pool4(y, 16)

    _repack(p1_ref, a2_ref, 20, 16, 32, 16)
    for i in range(NI):
        p2_ref[i, :, :] = conv_img(a2_ref, i, S2, 18,
                                   w2_ref, b2_ref, 32)

    _repack(p2_ref, a3_ref, 10, 8, 18, 32)
    for i in range(NI):
        p3_ref[i, :, :] = conv_img(a3_ref, i, S3, 10,
                                   w3_ref, b3_ref, 64)

    _repack(p3_ref, a4_ref, 5, 4, 10, 64)
    acc = None
    for dy in range(3):
        xc = jnp.concatenate(
            [jnp.concatenate(
                [a4_ref[i, pl.ds(dy * 6 + dx, S4), :] for dx in range(3)],
                axis=1) for i in range(NI)],
            axis=0)
        d = jnp.dot(xc, w4_ref[dy], preferred_element_type=jnp.float32)
        acc = d if acc is None else acc + d
    y = jnp.maximum(acc + b4_ref[...], 0.0)
    p4 = _pool4(y, 128)
    v = p4.reshape(NI, S4, 128) * wl_ref[...][None]
    o_ref[...] = (jnp.sum(v, axis=(1, 2), keepdims=True)
                  + bl_ref[0, 0])


def _prep(x):
    B = x.shape[0]
    xc = x.reshape(B, 40, 2, 32, 2).transpose(0, 1, 3, 2, 4).reshape(
        B, 40, 32, 4)
    xc = jnp.pad(xc, ((0, 0), (1, 1), (1, 1), (0, 0)))
    taps = [xc[:, dy:dy + 40, dx:dx + 32, :]
            for dy in range(3) for dx in range(3)]
    xi = jnp.concatenate(taps, axis=-1)
    return xi.reshape(B, 1280, 36).astype(jnp.bfloat16)


@jax.jit
def kernel(x, w1, b1, w2, b2, w3, b3, w4, b4, wl, bl):
    B = x.shape[0]
    assert B % NI == 0, B
    xi = _prep(x)
    out = pl.pallas_call(
        _cnn_kernel,
        out_shape=jax.ShapeDtypeStruct((B, 1, 1), jnp.float32),
        grid=(B // NI,),
        in_specs=[
            pl.BlockSpec((NI, S1, 36), lambda i: (i, 0, 0)),
            pl.BlockSpec((36, 64), lambda i: (0, 0)),
            pl.BlockSpec((1, 64), lambda i: (0, 0)),
            pl.BlockSpec((3, 192, 128), lambda i: (0, 0, 0)),
            pl.BlockSpec((1, 128), lambda i: (0, 0)),
            pl.BlockSpec((3, 384, 256), lambda i: (0, 0, 0)),
            pl.BlockSpec((1, 256), lambda i: (0, 0)),
            pl.BlockSpec((3, 768, 512), lambda i: (0, 0, 0)),
            pl.BlockSpec((1, 512), lambda i: (0, 0)),
            pl.BlockSpec((30, 128), lambda i: (0, 0)),
            pl.BlockSpec((1, 1), lambda i: (0, 0)),
        ],
        out_specs=pl.BlockSpec((NI, 1, 1), lambda i: (i, 0, 0)),
        scratch_shapes=[
            pltpu.VMEM((NI, S1, 16), jnp.float32),
            pltpu.VMEM((NI, R2, 64), jnp.bfloat16),
            pltpu.VMEM((NI, S2, 32), jnp.float32),
            pltpu.VMEM((NI, R3, 128), jnp.bfloat16),
            pltpu.VMEM((NI, S3, 64), jnp.float32),
            pltpu.VMEM((NI, R4, 256), jnp.bfloat16),
        ],
        compiler_params=pltpu.CompilerParams(
            dimension_semantics=("parallel",),
            vmem_limit_bytes=64 * 1024 * 1024),
    )(xi, w1, b1, w2, b2, w3, b3, w4, b4, wl, bl)
    return out.reshape(B)
